# Initial kernel scaffold; baseline (speedup 1.0000x reference)
import jax
import jax.numpy as jnp
from jax.experimental import pallas as pl


def kernel(attr, edge_attr, edge_index, W0_1, W1_1, W2_1, Wa_1, W0_2, W1_2, W2_2, Wa_2):
    raise NotImplementedError("write your pallas kernel here")


# speedup vs baseline: 1.1774x; 1.1774x over previous
import functools
import jax
import jax.numpy as jnp
from jax.experimental import pallas as pl

N = 10000
D = 128
H = 128
_BM = 1000


def _mm_body(x_ref, w_ref, o_ref):
    o_ref[...] = jnp.dot(x_ref[...], w_ref[...],
                         preferred_element_type=jnp.float32)


def _matmul(x, w):
    m, k = x.shape
    _, n = w.shape
    return pl.pallas_call(
        _mm_body,
        grid=(m // _BM,),
        in_specs=[
            pl.BlockSpec((_BM, k), lambda i: (i, 0)),
            pl.BlockSpec((k, n), lambda i: (0, 0)),
        ],
        out_specs=pl.BlockSpec((_BM, n), lambda i: (i, 0)),
        out_shape=jax.ShapeDtypeStruct((m, n), jnp.float32),
    )(x, w)


def _combine_body(zi_ref, s_ref, d_ref, o_ref):
    den = d_ref[:, 0:1]
    den = jnp.where(den > 0, den, 1.0)
    o_ref[...] = jnp.maximum(zi_ref[...] + s_ref[...] / den, 0.0)


def _combine(zi, s, den):
    return pl.pallas_call(
        _combine_body,
        grid=(N // _BM,),
        in_specs=[
            pl.BlockSpec((_BM, H), lambda i: (i, 0)),
            pl.BlockSpec((_BM, H), lambda i: (i, 0)),
            pl.BlockSpec((_BM, 1), lambda i: (i, 0)),
        ],
        out_specs=pl.BlockSpec((_BM, H), lambda i: (i, 0)),
        out_shape=jax.ShapeDtypeStruct((N, H), jnp.float32),
    )(zi, s, den)


def _layer(h, ea, src, dst, W0, W1, W2, Wa):
    wa_s = Wa[0, :H]
    wa_d = Wa[0, H:2 * H]
    coef = W0[0, 0] * Wa[0, 2 * H]
    u = jnp.zeros((h.shape[1], 128), jnp.float32)
    u = u.at[:, 0].set(W1.T @ wa_s).at[:, 1].set(W1.T @ wa_d)
    wbig = jnp.concatenate([W1.T, W2.T, u], axis=1)
    y = _matmul(h, wbig)
    z = y[:, :H]
    zi = y[:, H:2 * H]
    a_s = y[:, 2 * H]
    a_d = y[:, 2 * H + 1]
    x = a_s[src] + a_d[dst] + coef * ea
    e = jnp.where(x > 0, x, 0.01 * x)
    w = jnp.exp(e)
    den = jax.ops.segment_sum(w, dst, num_segments=N)
    s = jax.ops.segment_sum(w[:, None] * z[src], dst, num_segments=N)
    return _combine(zi, s, den[:, None])


def kernel(attr, edge_attr, edge_index, W0_1, W1_1, W2_1, Wa_1,
           W0_2, W1_2, W2_2, Wa_2):
    src = edge_index[0]
    dst = edge_index[1]
    ea = edge_attr[:, 0]
    h = _layer(attr, ea, src, dst, W0_1, W1_1, W2_1, Wa_1)
    h = _layer(h, ea, src, dst, W0_2, W1_2, W2_2, Wa_2)
    return h


# speedup vs baseline: 32.3121x; 27.4432x over previous
import functools
import jax
import jax.numpy as jnp
from jax import lax
from jax.experimental import pallas as pl
from jax.experimental.pallas import tpu as pltpu
from jax.experimental.pallas import tpu_sc as plsc

N = 10000
D = 128
H = 128
E = 320000

_NC = 2
_NS = 16
_NW = _NC * _NS
_L = 16

_EPT = E // _NW
_NB = 5
_GPT = _EPT // _L
_TOUT = _GPT // _NB
_RPT = 624
_RCH = 24

_BM = 1000

_SC_PARAMS = pltpu.CompilerParams(needs_layout_passes=False)
_SC_MESH = plsc.VectorSubcoreMesh(core_axis_name="c", subcore_axis_name="s")


def _matmul_body(x_ref, w_ref, o_ref):
    o_ref[...] = jnp.dot(x_ref[...], w_ref[...],
                         preferred_element_type=jnp.float32)


def _matmul(x, w):
    m, k = x.shape
    _, n = w.shape
    return pl.pallas_call(
        _matmul_body,
        grid=(m // _BM,),
        in_specs=[
            pl.BlockSpec((_BM, k), lambda i: (i, 0)),
            pl.BlockSpec((k, n), lambda i: (0, 0)),
        ],
        out_specs=pl.BlockSpec((_BM, n), lambda i: (i, 0)),
        out_shape=jax.ShapeDtypeStruct((m, n), jnp.float32),
    )(x, w)


def _combine_mm_body(zi_ref, sp_ref, dp_ref, w_ref, o_ref):
    den = dp_ref[:, 0:1] + dp_ref[:, 1:2]
    den = jnp.where(den > 0, den, 1.0)
    h = jnp.maximum(zi_ref[...] + (sp_ref[0] + sp_ref[1]) / den, 0.0)
    o_ref[...] = jnp.dot(h, w_ref[...], preferred_element_type=jnp.float32)


def _combine_mm(zi, sp, dp, w):
    n = w.shape[1]
    return pl.pallas_call(
        _combine_mm_body,
        grid=(N // _BM,),
        in_specs=[
            pl.BlockSpec((_BM, H), lambda i: (i, 0)),
            pl.BlockSpec((2, _BM, H), lambda i: (0, i, 0)),
            pl.BlockSpec((_BM, 2), lambda i: (i, 0)),
            pl.BlockSpec((H, n), lambda i: (0, 0)),
        ],
        out_specs=pl.BlockSpec((_BM, n), lambda i: (i, 0)),
        out_shape=jax.ShapeDtypeStruct((N, n), jnp.float32),
    )(zi, sp, dp, w)


def _combine_body(zi_ref, sp_ref, dp_ref, o_ref):
    den = dp_ref[:, 0:1] + dp_ref[:, 1:2]
    den = jnp.where(den > 0, den, 1.0)
    o_ref[...] = jnp.maximum(zi_ref[...] + (sp_ref[0] + sp_ref[1]) / den, 0.0)


def _combine(zi, sp, dp):
    return pl.pallas_call(
        _combine_body,
        grid=(N // _BM,),
        in_specs=[
            pl.BlockSpec((_BM, H), lambda i: (i, 0)),
            pl.BlockSpec((2, _BM, H), lambda i: (0, i, 0)),
            pl.BlockSpec((_BM, 2), lambda i: (i, 0)),
        ],
        out_specs=pl.BlockSpec((_BM, H), lambda i: (i, 0)),
        out_shape=jax.ShapeDtypeStruct((N, H), jnp.float32),
    )(zi, sp, dp)


@functools.partial(
    pl.kernel,
    out_type=[
        jax.ShapeDtypeStruct((E,), jnp.float32),
        jax.ShapeDtypeStruct((_NC, 1, N), jnp.float32),
    ],
    mesh=_SC_MESH,
    compiler_params=_SC_PARAMS,
    scratch_types=[
        pltpu.VMEM((_EPT,), jnp.int32),
        pltpu.VMEM((_EPT,), jnp.int32),
        pltpu.VMEM((_EPT,), jnp.float32),
        pltpu.VMEM((_EPT,), jnp.float32),
        pltpu.VMEM((N,), jnp.float32),
        pltpu.VMEM((N,), jnp.float32),
        pltpu.VMEM((_L,), jnp.float32),
        pltpu.VMEM((_NB, _L), jnp.float32),
        pltpu.VMEM((1, N), jnp.float32),
        pltpu.VMEM_SHARED((N,), jnp.float32),
        pltpu.SemaphoreType.DMA((_NB,)),
    ],
)
def _edge_weights(src_hbm, dst_hbm, ea_hbm, as_hbm, ad_hbm, coef_hbm,
                  w_out, den_out,
                  src_v, dst_v, ea_v, w_v, as_v, ad_v, coef_v,
                  wbuf, dden, den_sp, dsem):
    c = lax.axis_index("c")
    s = lax.axis_index("s")
    wid = c * _NS + s
    ebase = wid * _EPT

    pltpu.sync_copy(src_hbm.at[pl.ds(ebase, _EPT)], src_v)
    pltpu.sync_copy(dst_hbm.at[pl.ds(ebase, _EPT)], dst_v)
    pltpu.sync_copy(ea_hbm.at[pl.ds(ebase, _EPT)], ea_v)
    pltpu.sync_copy(as_hbm, as_v)
    pltpu.sync_copy(ad_hbm, ad_v)
    pltpu.sync_copy(coef_hbm, coef_v)

    zero = jnp.zeros((_L,), jnp.float32)

    @pl.when(s == 0)
    def _():
        def _zden(r, _):
            dden[0, pl.ds(r * _L, _L)] = zero
            return 0
        lax.fori_loop(0, N // _L, _zden, 0)
        pltpu.sync_copy(dden.at[0], den_sp)

    plsc.subcore_barrier()

    coefv = coef_v[...]

    def _outer(t, _):
        for b in range(_NB):
            g = t * _NB + b
            srcv = src_v[pl.ds(g * _L, _L)]
            dstv = dst_v[pl.ds(g * _L, _L)]
            tv = ea_v[pl.ds(g * _L, _L)]
            x = (plsc.load_gather(as_v, [srcv])
                 + plsc.load_gather(ad_v, [dstv]) + coefv * tv)
            x = jnp.where(x > 0, x, 0.01 * x)
            w = jnp.exp(x)
            w_v[pl.ds(g * _L, _L)] = w

            @pl.when(t > 0)
            def _():
                pltpu.make_async_copy(wbuf.at[b], den_sp.at[dstv],
                                      dsem.at[b]).wait()

            wbuf[b, ...] = w
            pltpu.async_copy(wbuf.at[b], den_sp.at[dstv], dsem.at[b],
                             add=True)
        return 0

    lax.fori_loop(0, _TOUT, _outer, 0)

    dstv0 = dst_v[pl.ds(0, _L)]
    for b in range(_NB):
        pltpu.make_async_copy(wbuf.at[b], den_sp.at[dstv0],
                              dsem.at[b]).wait()

    pltpu.sync_copy(w_v, w_out.at[pl.ds(ebase, _EPT)])

    plsc.subcore_barrier()

    @pl.when(s == 0)
    def _():
        pltpu.sync_copy(den_sp, dden.at[0])
        pltpu.sync_copy(dden, den_out.at[c])


@functools.partial(
    pl.kernel,
    out_type=jax.ShapeDtypeStruct((_NC, N, H), jnp.float32),
    mesh=_SC_MESH,
    compiler_params=_SC_PARAMS,
    scratch_types=[
        pltpu.VMEM((_EPT,), jnp.int32),
        pltpu.VMEM((_EPT,), jnp.int32),
        pltpu.VMEM((_NB, _L), jnp.float32),
        pltpu.VMEM((_NB, _L, H), jnp.float32),
        pltpu.VMEM((_NB, _L, H), jnp.float32),
        pltpu.VMEM((_RCH, H), jnp.float32),
        pltpu.VMEM_SHARED((N, H), jnp.float32),
        pltpu.SemaphoreType.DMA((_NB,)),
        pltpu.SemaphoreType.DMA((_NB,)),
        pltpu.SemaphoreType.DMA((_NB,)),
    ],
)
def _edge_scatter(z_hbm, src_hbm, dst_hbm, w_hbm, s_out,
                  src_v, dst_v, wring, rbuf, obuf, stage,
                  s_sp, gsem, wsem, ssem):
    c = lax.axis_index("c")
    s = lax.axis_index("s")
    wid = c * _NS + s
    ebase = wid * _EPT

    pltpu.sync_copy(src_hbm.at[pl.ds(ebase, _EPT)], src_v)
    pltpu.sync_copy(dst_hbm.at[pl.ds(ebase, _EPT)], dst_v)

    zero = jnp.zeros((_L,), jnp.float32)

    def _zrow(r, _):
        for j in range(H // _L):
            stage[r, pl.ds(j * _L, _L)] = zero
        return 0

    lax.fori_loop(0, _RCH, _zrow, 0)

    row0 = s * _RPT
    for k in range(_RPT // _RCH):
        pltpu.sync_copy(stage, s_sp.at[pl.ds(row0 + k * _RCH, _RCH)])

    @pl.when(s == _NS - 1)
    def _():
        pltpu.sync_copy(stage.at[pl.ds(0, _L)], s_sp.at[pl.ds(9984, _L)])

    plsc.subcore_barrier()

    for b in range(_NB):
        srcv0 = src_v[pl.ds(b * _L, _L)]
        pltpu.async_copy(z_hbm.at[srcv0], rbuf.at[b], gsem.at[b])
        pltpu.async_copy(w_hbm.at[pl.ds(ebase + b * _L, _L)],
                         wring.at[b], wsem.at[b])

    def _outer(t, _):
        for b in range(_NB):
            g = t * _NB + b
            srcv = src_v[pl.ds(g * _L, _L)]
            dstv = dst_v[pl.ds(g * _L, _L)]
            pltpu.make_async_copy(z_hbm.at[srcv], rbuf.at[b],
                                  gsem.at[b]).wait()
            pltpu.make_async_copy(w_hbm.at[pl.ds(ebase, _L)],
                                  wring.at[b], wsem.at[b]).wait()
            wv = wring[b, ...]

            @pl.when(t > 0)
            def _():
                pltpu.make_async_copy(obuf.at[b], s_sp.at[dstv],
                                      ssem.at[b]).wait()

            for i in range(_L):
                wvi = jnp.full((_L,), wv[i])
                for j in range(H // _L):
                    obuf[b, i, pl.ds(j * _L, _L)] = (
                        rbuf[b, i, pl.ds(j * _L, _L)] * wvi)

            @pl.when(t < _TOUT - 1)
            def _():
                srcv2 = src_v[pl.ds((g + _NB) * _L, _L)]
                pltpu.async_copy(z_hbm.at[srcv2], rbuf.at[b], gsem.at[b])
                pltpu.async_copy(w_hbm.at[pl.ds(ebase + (g + _NB) * _L, _L)],
                                 wring.at[b], wsem.at[b])

            pltpu.async_copy(obuf.at[b], s_sp.at[dstv], ssem.at[b],
                             add=True)
        return 0

    lax.fori_loop(0, _TOUT, _outer, 0)

    dstv0 = dst_v[pl.ds(0, _L)]
    for b in range(_NB):
        pltpu.make_async_copy(obuf.at[b], s_sp.at[dstv0],
                              ssem.at[b]).wait()

    plsc.subcore_barrier()

    for k in range(_RPT // _RCH):
        pltpu.sync_copy(s_sp.at[pl.ds(row0 + k * _RCH, _RCH)], stage)
        pltpu.sync_copy(stage, s_out.at[c, pl.ds(row0 + k * _RCH, _RCH)])

    @pl.when(s == _NS - 1)
    def _():
        pltpu.sync_copy(s_sp.at[pl.ds(9984, _L)], stage.at[pl.ds(0, _L)])
        pltpu.sync_copy(stage.at[pl.ds(0, _L)], s_out.at[c, pl.ds(9984, _L)])


def _make_wbig(W0, W1, W2, Wa):
    wa_s = Wa[0, :H]
    wa_d = Wa[0, H:2 * H]
    coef = W0[0, 0] * Wa[0, 2 * H]
    u = jnp.zeros((W1.shape[1], 128), jnp.float32)
    u = u.at[:, 0].set(W1.T @ wa_s).at[:, 1].set(W1.T @ wa_d)
    wbig = jnp.concatenate([W1.T, W2.T, u], axis=1)
    return wbig, jnp.full((_L,), coef, jnp.float32)


def kernel(attr, edge_attr, edge_index, W0_1, W1_1, W2_1, Wa_1,
           W0_2, W1_2, W2_2, Wa_2):
    src = edge_index[0].astype(jnp.int32)
    dst = edge_index[1].astype(jnp.int32)
    ea = edge_attr[:, 0]

    wbig1, coef1 = _make_wbig(W0_1, W1_1, W2_1, Wa_1)
    wbig2, coef2 = _make_wbig(W0_2, W1_2, W2_2, Wa_2)

    y1 = _matmul(attr, wbig1)
    z1, zi1 = y1[:, :H], y1[:, H:2 * H]
    as1, ad1 = y1[:, 2 * H], y1[:, 2 * H + 1]
    w1, dp1 = _edge_weights(src, dst, ea, as1, ad1, coef1)
    sp1 = _edge_scatter(z1, src, dst, w1)

    y2 = _combine_mm(zi1, sp1, dp1.reshape(_NC, N).T, wbig2)
    z2, zi2 = y2[:, :H], y2[:, H:2 * H]
    as2, ad2 = y2[:, 2 * H], y2[:, 2 * H + 1]
    w2, dp2 = _edge_weights(src, dst, ea, as2, ad2, coef2)
    sp2 = _edge_scatter(z2, src, dst, w2)

    return _combine(zi2, sp2, dp2.reshape(_NC, N).T)
